# baseline (speedup 1.0000x reference)
import functools

import jax
import jax.numpy as jnp
from jax import lax
from jax.experimental import pallas as pl
from jax.experimental.pallas import tpu as pltpu
from jax.experimental.pallas import tpu_sc as plsc

B = 4
IN_CH = 3
HID = 128
RES = 64
NLAYERS = 2
K = 8192
CDIM = 32
HW = 224
COMMIT = 0.25
ORTHO_W = 1.0

N = B * 56 * 56
_PREC = lax.Precision.HIGHEST

ROWS_BLK = 1792
K_CHUNK = 2048


def _conv(x, w, b, stride, pad):
    y = lax.conv_general_dilated(
        x, w, (stride, stride), ((pad, pad), (pad, pad)),
        dimension_numbers=('NCHW', 'OIHW', 'NCHW'))
    return y + b[None, :, None, None]


def _deconv(x, w, b):
    y = lax.conv_transpose(x, w, (2, 2), 'SAME',
                           dimension_numbers=('NCHW', 'OIHW', 'NCHW'))
    return y + b[None, :, None, None]


def _residual_stack(h, w3, b3, w1, b1):
    for i in range(NLAYERS):
        r = jax.nn.relu(h)
        r = jax.nn.relu(_conv(r, w3[i], b3[i], 1, 1))
        r = _conv(r, w1[i], b1[i], 1, 0)
        h = h + r
    return jax.nn.relu(h)


def _norm32(p):
    a = p * p
    s = a[:, 0:8] + a[:, 8:16]
    s = s + a[:, 16:24]
    s = s + a[:, 24:32]
    t = s[:, 0:4] + s[:, 4:8]
    t = t[:, 0:2] + t[:, 2:4]
    t = t[:, 0:1] + t[:, 1:2]
    return jnp.sqrt(t)


def _latn_body(p_ref, latn_ref):
    p = p_ref[...]
    latn_ref[...] = p / (_norm32(p) + 1e-12)


def _latn_call(p):
    return pl.pallas_call(
        _latn_body,
        grid=(N // ROWS_BLK,),
        in_specs=[pl.BlockSpec((ROWS_BLK, CDIM), lambda i: (i, 0))],
        out_specs=pl.BlockSpec((ROWS_BLK, CDIM), lambda i: (i, 0)),
        out_shape=jax.ShapeDtypeStruct((N, CDIM), jnp.float32),
    )(p)


def _cbn_body(cb_ref, cbn_ref):
    cb = cb_ref[...]
    cbn_ref[...] = cb / (_norm32(cb) + 1e-12)


def _cbn_call(codebook):
    return pl.pallas_call(
        _cbn_body,
        out_shape=jax.ShapeDtypeStruct((K, CDIM), jnp.float32),
    )(codebook)


def _argmax_body(latn_ref, cbn_ref, ind_ref):
    latn = latn_ref[...]

    def step(c, carry):
        best_s, best_i = carry
        cbc = cbn_ref[pl.ds(c * K_CHUNK, K_CHUNK), :]
        s = lax.dot_general(latn.astype(jnp.bfloat16), cbc.astype(jnp.bfloat16),
                            (((1,), (1,)), ((), ())),
                            preferred_element_type=jnp.float32)
        m = jnp.max(s, axis=1)
        iota = lax.broadcasted_iota(jnp.int32, s.shape, 1)
        idx = jnp.min(jnp.where(s == m[:, None], iota, K), axis=1) + c * K_CHUNK
        upd = m > best_s
        return jnp.where(upd, m, best_s), jnp.where(upd, idx, best_i)

    init = (jnp.full((ROWS_BLK,), -jnp.inf, jnp.float32),
            jnp.zeros((ROWS_BLK,), jnp.int32))
    _, best_i = lax.fori_loop(0, K // K_CHUNK, step, init)
    ind_ref[...] = best_i.reshape(1, 1, ROWS_BLK)


def _argmax_call(latn, cbn):
    return pl.pallas_call(
        _argmax_body,
        grid=(N // ROWS_BLK,),
        in_specs=[
            pl.BlockSpec((ROWS_BLK, CDIM), lambda i: (i, 0)),
            pl.BlockSpec((K, CDIM), lambda i: (0, 0)),
        ],
        out_specs=pl.BlockSpec((1, 1, ROWS_BLK), lambda i: (i, 0, 0)),
        out_shape=jax.ShapeDtypeStruct((N // ROWS_BLK, 1, ROWS_BLK), jnp.int32),
    )(latn, cbn)


_NC = 2
_NS = 16
_NW = _NC * _NS
BPW = N // _NW
BPAD = 400
_GCHUNK = 128


_GD = 128


def _sc_gather_body(cb_hbm, idx_hbm, quant_hbm, counts_hbm, idx_v, rows_v, cnt_v, sem):
    wid = lax.axis_index("c") * _NS + lax.axis_index("s")
    base = wid * BPW
    idx_v[pl.ds(384, 16)] = jnp.zeros((16,), jnp.int32)
    pltpu.sync_copy(idx_hbm.at[pl.ds(base, BPW)], idx_v.at[pl.ds(0, BPW)])
    for ch in range(4):
        off = ch * _GCHUNK
        ln = _GCHUNK if ch < 3 else BPAD - 3 * _GCHUNK
        pltpu.async_copy(cb_hbm.at[idx_v.at[pl.ds(off, ln)]],
                         rows_v.at[pl.ds(off, ln)], sem).wait()
    pltpu.sync_copy(rows_v.at[pl.ds(0, BPW)], quant_hbm.at[pl.ds(base, BPW)])
    zeros16 = jnp.zeros((16,), jnp.float32)

    def zero_step(j, carry):
        cnt_v[pl.ds(j * 16, 16)] = zeros16
        return carry

    lax.fori_loop(0, K // 16, zero_step, 0)
    ones16 = jnp.ones((16,), jnp.float32)
    lane = lax.iota(jnp.int32, 16)

    def cnt_step(j, carry):
        idxs = idx_v[pl.ds(j * 16, 16)]
        mask = (j * 16 + lane) < BPW
        plsc.addupdate_scatter(cnt_v, [idxs], ones16, mask=mask)
        return carry

    lax.fori_loop(0, BPAD // 16, cnt_step, 0)
    pltpu.sync_copy(cnt_v, counts_hbm.at[wid])


@functools.lru_cache(maxsize=1)
def _sc_gather_kernel():
    return pl.kernel(
        _sc_gather_body,
        out_type=(jax.ShapeDtypeStruct((N, _GD), jnp.float32),
                  jax.ShapeDtypeStruct((_NW, K), jnp.float32)),
        mesh=plsc.VectorSubcoreMesh(core_axis_name="c", subcore_axis_name="s"),
        scratch_types=[
            pltpu.VMEM((BPAD,), jnp.int32),
            pltpu.VMEM((BPAD, _GD), jnp.float32),
            pltpu.VMEM((K,), jnp.float32),
            pltpu.SemaphoreType.DMA,
        ],
        compiler_params=pltpu.CompilerParams(needs_layout_passes=False),
    )


def _sc_gather(codebook, embed_flat):
    cb_pad = jnp.pad(codebook, ((0, 0), (0, _GD - CDIM)))
    return _sc_gather_kernel()(cb_pad, embed_flat)


def _post_body(quant_ref, latn_ref, cbn_ref, pow_ref, pob_ref, cnt_ref,
               q_ref, scal_ref):
    quant = quant_ref[...][:, :CDIM]
    q_ref[...] = lax.dot_general(quant.astype(jnp.bfloat16),
                                 pow_ref[...].astype(jnp.bfloat16),
                                 (((1,), (1,)), ((), ())),
                                 preferred_element_type=jnp.float32) + pob_ref[...]
    d = quant - latn_ref[...]
    commit = jnp.sum(d * d) / (N * CDIM)
    cbn = cbn_ref[...]
    g = lax.dot_general(cbn, cbn, (((0,), (0,)), ((), ())), precision=_PREC)
    ortho = (jnp.sum(g * g) - K) / (K * (K - 1.0))
    counts = jnp.sum(cnt_ref[...], axis=0)
    probs = counts / N
    perp = jnp.exp(-jnp.sum(probs * jnp.log(probs + 1e-10)))
    i = lax.broadcasted_iota(jnp.int32, (1, 128), 1)
    scal_ref[...] = jnp.where(i == 0, commit,
                              jnp.where(i == 1, ortho,
                                        jnp.where(i == 2, perp, 0.0)))


def _post_call(quant, latn, cbn, proj_out_w, proj_out_b, counts32):
    return pl.pallas_call(
        _post_body,
        out_shape=[
            jax.ShapeDtypeStruct((N, HID), jnp.float32),
            jax.ShapeDtypeStruct((1, 128), jnp.float32),
        ],
    )(quant, latn, cbn, proj_out_w, proj_out_b.reshape(1, HID), counts32)


def _recon_body(a_ref, b_ref, out_ref):
    d = a_ref[...] - b_ref[...]
    s = jnp.sum(d * d) / (B * IN_CH * HW * HW)
    out_ref[...] = s * jnp.ones((1, 128), jnp.float32)


def _recon_call(x_recon, x):
    flat = B * IN_CH * HW * HW // 128
    return pl.pallas_call(
        _recon_body,
        out_shape=jax.ShapeDtypeStruct((1, 128), jnp.float32),
    )(x_recon.reshape(flat, 128), x.reshape(flat, 128))


def kernel(x, enc_w1, enc_b1, enc_w2, enc_b2, enc_w3, enc_b3, enc_res_w3,
           enc_res_b3, enc_res_w1, enc_res_b1, proj_in_w, proj_in_b, codebook,
           proj_out_w, proj_out_b, dec_w1, dec_b1, dec_res_w3, dec_res_b3,
           dec_res_w1, dec_res_b1, dec_tw1, dec_tb1, dec_tw2, dec_tb2):
    h = jax.nn.relu(_conv(x, enc_w1, enc_b1, 2, 1))
    h = jax.nn.relu(_conv(h, enc_w2, enc_b2, 2, 1))
    h = _conv(h, enc_w3, enc_b3, 1, 1)
    z = _residual_stack(h, enc_res_w3, enc_res_b3, enc_res_w1, enc_res_b1)
    bsz, ch, hh, ww = z.shape
    lat3 = z.transpose(0, 2, 3, 1).reshape(bsz, hh * ww, ch)
    p = (lat3 @ proj_in_w.T + proj_in_b).reshape(N, CDIM)

    latn = _latn_call(p)
    cbn = _cbn_call(codebook)
    embed_blk = _argmax_call(latn, cbn)
    embed_flat = embed_blk.reshape(N)
    quant, counts32 = _sc_gather(codebook, embed_flat)
    q, scal = _post_call(quant, latn, cbn, proj_out_w, proj_out_b, counts32)
    commit_loss = scal[0, 0]
    ortho_loss = scal[0, 1]
    perplexity = scal[0, 2]

    q_img = q.reshape(bsz, hh, ww, HID).transpose(0, 3, 1, 2)
    h2 = _conv(q_img, dec_w1, dec_b1, 1, 1)
    h2 = _residual_stack(h2, dec_res_w3, dec_res_b3, dec_res_w1, dec_res_b1)
    h2 = jax.nn.relu(_deconv(h2, dec_tw1, dec_tb1))
    x_recon = _deconv(h2, dec_tw2, dec_tb2)

    recon_loss = _recon_call(x_recon, x)[0, 0]
    vq_loss = COMMIT * commit_loss + ORTHO_W * ortho_loss
    loss = recon_loss + vq_loss
    embed_ind = embed_flat.reshape(bsz, hh * ww)
    return (q_img, x_recon, loss, recon_loss, vq_loss, commit_loss,
            ortho_loss, embed_ind, perplexity)
